# baseline (speedup 1.0000x reference)
import functools

import jax
import jax.numpy as jnp
from jax.experimental import pallas as pl
from jax.experimental.pallas import tpu as pltpu

_DILATIONS = (12, 24, 36)
_CH = 4


def _fused_kernel(x_ref, b0w_ref, b0o_ref, dilw_ref, dilo_ref,
                  poolw_ref, poolo_ref, projw_ref, projo_ref,
                  headw_ref, heado_ref, clsw_ref, clsb_ref,
                  o_ref, xpa, xpb, pb0, pb1, pb2, *, H, W, P, dils):
    cin = x_ref.shape[-1]
    C = b0w_ref.shape[-1]
    CP = clsw_ref.shape[-1]
    f32 = jnp.float32
    bf16 = jnp.bfloat16
    CH = _CH
    M = CH * W
    QA, QB = 40, 44

    xpa[...] = jnp.zeros_like(xpa)
    xpb[...] = jnp.zeros_like(xpb)
    pb0[...] = jnp.zeros_like(pb0)
    pb1[...] = jnp.zeros_like(pb1)
    pb2[...] = jnp.zeros_like(pb2)
    xi = x_ref[0]
    xpa[P:P + H, QA:QA + W, :] = xi
    xpb[P:P + H, QB:QB + W, :] = xi

    interior = xpa[P:P + H, QA:QA + W, :].reshape(H * W, cin)
    ones = jnp.ones((8, H * W), bf16)
    mean = (jnp.dot(ones, interior, preferred_element_type=f32)[0:1]
            * (1.0 / (H * W)))
    pooled = jnp.dot(mean.astype(bf16), poolw_ref[...],
                     preferred_element_type=f32)
    pooled = jnp.maximum(pooled + poolo_ref[...], 0.0)
    pool_proj = jnp.dot(pooled.astype(bf16), projw_ref[4],
                        preferred_element_type=f32)


    for c in range(H // CH):
        r0 = c * CH
        xs = xpa[P + r0:P + r0 + CH, QA:QA + W, :].reshape(M, cin)
        b0 = jnp.dot(xs, b0w_ref[...], preferred_element_type=f32)
        b0 = jnp.maximum(b0 + b0o_ref[...], 0.0)
        pacc = jnp.dot(b0.astype(bf16), projw_ref[0],
                       preferred_element_type=f32) + pool_proj
        for i, d in enumerate(dils):
            conv = None
            for kh in range(3):
                dh = (kh - 1) * d
                lo = max(0, -dh)
                hi = H - max(0, dh)
                if r0 + CH <= lo or r0 >= hi:
                    continue
                for kw in range(3):
                    dw = (kw - 1) * d
                    if (QA + dw) % 8 == 0:
                        patch = xpa[P + dh + r0:P + dh + r0 + CH,
                                    QA + dw:QA + dw + W, :]
                    else:
                        patch = xpb[P + dh + r0:P + dh + r0 + CH,
                                    QB + dw:QB + dw + W, :]
                    patch = patch.reshape(M, cin)
                    t = jnp.dot(patch, dilw_ref[i * 9 + kh * 3 + kw],
                                preferred_element_type=f32)
                    conv = t if conv is None else conv + t
            bi = jnp.maximum(conv + dilo_ref[i], 0.0)
            pacc = pacc + jnp.dot(bi.astype(bf16), projw_ref[i + 1],
                                  preferred_element_type=f32)
        proj = jnp.maximum(pacc + projo_ref[...], 0.0).astype(bf16)
        pr = proj.reshape(CH, W, C)
        pb1[1 + r0:1 + r0 + CH, :, :] = pr
        pb0[1 + r0:1 + r0 + CH, 1:W, :] = pr[:, :W - 1, :]
        pb2[1 + r0:1 + r0 + CH, 0:W - 1, :] = pr[:, 1:, :]

    for c in range(H // CH):
        r0 = c * CH
        hacc = None
        for kh in range(3):
            for kw, pb in enumerate((pb0, pb1, pb2)):
                patch = pb[r0 + kh:r0 + kh + CH, :, :].reshape(M, C)
                t = jnp.dot(patch, headw_ref[kh * 3 + kw],
                            preferred_element_type=f32)
                hacc = t if hacc is None else hacc + t
        h = jnp.maximum(hacc + heado_ref[...], 0.0).astype(bf16)
        logits = jax.lax.dot_general(clsw_ref[...], h,
                                     (((0,), (1,)), ((), ())),
                                     preferred_element_type=f32)
        nco = o_ref.shape[1]
        o_ref[0, :, r0 * W:r0 * W + M] = (logits + clsb_ref[...])[:nco]


def kernel(b0_w, b0_scale, b0_offset, dil_w, dil_scale, dil_offset,
           pool_w, pool_scale, pool_offset, proj_w, proj_scale, proj_offset,
           head_w, head_scale, head_offset, cls_w, cls_b, x):
    N, cin, H, W = x.shape
    C = b0_w.shape[-1]
    P = max(_DILATIONS)
    nc = cls_w.shape[1]
    CP = max(32, ((nc + 7) // 8) * 8)
    HW = H * W
    bf = jnp.bfloat16

    xh = jnp.transpose(x, (0, 2, 3, 1)).astype(bf)
    Hp, Wp = H + 2 * P, W + 2 * P

    b0w = (b0_w * b0_scale).astype(bf)
    dilw = (dil_w * dil_scale[:, None, None]).reshape(9 * len(_DILATIONS),
                                                     cin, C).astype(bf)
    poolw = (pool_w * pool_scale).astype(bf)
    projw = (proj_w * proj_scale).astype(bf)
    headw = (head_w * head_scale).reshape(9, C, C).astype(bf)
    clsw = jnp.pad(cls_w, ((0, 0), (0, CP - nc))).astype(bf)
    clsb = jnp.pad(cls_b, ((0, 0), (0, CP - nc))).reshape(CP, 1)

    def const(*shape):
        nd = len(shape)
        return pl.BlockSpec(shape, lambda n, _nd=nd: (0,) * _nd)

    out = pl.pallas_call(
        functools.partial(_fused_kernel, H=H, W=W, P=P, dils=_DILATIONS),
        out_shape=jax.ShapeDtypeStruct((N, nc, HW), jnp.float32),
        grid=(N,),
        in_specs=[
            pl.BlockSpec((1, H, W, cin), lambda n: (n, 0, 0, 0)),
            const(cin, C), const(1, C),
            const(9 * len(_DILATIONS), cin, C), const(len(_DILATIONS), 1, C),
            const(cin, C), const(1, C),
            const(5, C, C), const(1, C),
            const(9, C, C), const(1, C),
            const(cin, CP), const(CP, 1),
        ],
        out_specs=pl.BlockSpec((1, nc, HW), lambda n: (n, 0, 0)),
        scratch_shapes=[
            pltpu.VMEM((Hp, 40 + W + P, cin), bf),
            pltpu.VMEM((Hp, 44 + W + P, cin), bf),
            pltpu.VMEM((H + 2, W, C), bf),
            pltpu.VMEM((H + 2, W, C), bf),
            pltpu.VMEM((H + 2, W, C), bf),
        ],
        compiler_params=pltpu.CompilerParams(
            dimension_semantics=("parallel",),
            vmem_limit_bytes=100 * 1024 * 1024),
    )(xh, b0w, b0_offset, dilw, dil_offset, poolw, pool_offset,
      projw, proj_offset, headw, head_offset, clsw, clsb)
    return out.reshape(N, nc, H, W)


# speedup vs baseline: 1.1577x; 1.1577x over previous
import functools

import jax
import jax.numpy as jnp
from jax.experimental import pallas as pl
from jax.experimental.pallas import tpu as pltpu

_DILATIONS = (12, 24, 36)
_CH = 8


def _fused_kernel(x_ref, b0w_ref, b0o_ref, dilw_ref, dilo_ref,
                  poolw_ref, poolo_ref, projw_ref, projo_ref,
                  headw_ref, heado_ref, clsw_ref, clsb_ref,
                  o_ref, xpa, xpb, pb0, pb1, pb2, *, H, W, P, dils):
    cin = x_ref.shape[-1]
    C = b0w_ref.shape[-1]
    CP = clsw_ref.shape[-1]
    f32 = jnp.float32
    bf16 = jnp.bfloat16
    CH = _CH
    M = CH * W
    QA, QB = 40, 44

    xpa[...] = jnp.zeros_like(xpa)
    xpb[...] = jnp.zeros_like(xpb)
    pb0[...] = jnp.zeros_like(pb0)
    pb1[...] = jnp.zeros_like(pb1)
    pb2[...] = jnp.zeros_like(pb2)
    xi = x_ref[0]
    xpa[P:P + H, QA:QA + W, :] = xi
    xpb[P:P + H, QB:QB + W, :] = xi

    interior = xpa[P:P + H, QA:QA + W, :].reshape(H * W, cin)
    ones = jnp.ones((8, H * W), bf16)
    mean = (jnp.dot(ones, interior, preferred_element_type=f32)[0:1]
            * (1.0 / (H * W)))
    pooled = jnp.dot(mean.astype(bf16), poolw_ref[...],
                     preferred_element_type=f32)
    pooled = jnp.maximum(pooled + poolo_ref[...], 0.0)
    pool_proj = jnp.dot(pooled.astype(bf16), projw_ref[4],
                        preferred_element_type=f32)


    for c in range(H // CH):
        r0 = c * CH
        xs = xpa[P + r0:P + r0 + CH, QA:QA + W, :].reshape(M, cin)
        b0 = jnp.dot(xs, b0w_ref[...], preferred_element_type=f32)
        b0 = jnp.maximum(b0 + b0o_ref[...], 0.0)
        pacc = jnp.dot(b0.astype(bf16), projw_ref[0],
                       preferred_element_type=f32) + pool_proj
        for i, d in enumerate(dils):
            conv = None
            for kh in range(3):
                dh = (kh - 1) * d
                lo = max(0, -dh)
                hi = H - max(0, dh)
                if r0 + CH <= lo or r0 >= hi:
                    continue
                for kw in range(3):
                    dw = (kw - 1) * d
                    if (QA + dw) % 8 == 0:
                        patch = xpa[P + dh + r0:P + dh + r0 + CH,
                                    QA + dw:QA + dw + W, :]
                    else:
                        patch = xpb[P + dh + r0:P + dh + r0 + CH,
                                    QB + dw:QB + dw + W, :]
                    patch = patch.reshape(M, cin)
                    t = jnp.dot(patch, dilw_ref[i * 9 + kh * 3 + kw],
                                preferred_element_type=f32)
                    conv = t if conv is None else conv + t
            bi = jnp.maximum(conv + dilo_ref[i], 0.0)
            pacc = pacc + jnp.dot(bi.astype(bf16), projw_ref[i + 1],
                                  preferred_element_type=f32)
        proj = jnp.maximum(pacc + projo_ref[...], 0.0).astype(bf16)
        pr = proj.reshape(CH, W, C)
        pb1[1 + r0:1 + r0 + CH, :, :] = pr
        pb0[1 + r0:1 + r0 + CH, 1:W, :] = pr[:, :W - 1, :]
        pb2[1 + r0:1 + r0 + CH, 0:W - 1, :] = pr[:, 1:, :]

    for c in range(H // CH):
        r0 = c * CH
        hacc = None
        for kh in range(3):
            for kw, pb in enumerate((pb0, pb1, pb2)):
                patch = pb[r0 + kh:r0 + kh + CH, :, :].reshape(M, C)
                t = jnp.dot(patch, headw_ref[kh * 3 + kw],
                            preferred_element_type=f32)
                hacc = t if hacc is None else hacc + t
        h = jnp.maximum(hacc + heado_ref[...], 0.0).astype(bf16)
        logits = jax.lax.dot_general(clsw_ref[...], h,
                                     (((0,), (1,)), ((), ())),
                                     preferred_element_type=f32)
        nco = o_ref.shape[1]
        o_ref[0, :, r0 * W:r0 * W + M] = (logits + clsb_ref[...])[:nco]


def kernel(b0_w, b0_scale, b0_offset, dil_w, dil_scale, dil_offset,
           pool_w, pool_scale, pool_offset, proj_w, proj_scale, proj_offset,
           head_w, head_scale, head_offset, cls_w, cls_b, x):
    N, cin, H, W = x.shape
    C = b0_w.shape[-1]
    P = max(_DILATIONS)
    nc = cls_w.shape[1]
    CP = max(32, ((nc + 7) // 8) * 8)
    HW = H * W
    bf = jnp.bfloat16

    xh = jnp.transpose(x, (0, 2, 3, 1)).astype(bf)
    Hp, Wp = H + 2 * P, W + 2 * P

    b0w = (b0_w * b0_scale).astype(bf)
    dilw = (dil_w * dil_scale[:, None, None]).reshape(9 * len(_DILATIONS),
                                                     cin, C).astype(bf)
    poolw = (pool_w * pool_scale).astype(bf)
    projw = (proj_w * proj_scale).astype(bf)
    headw = (head_w * head_scale).reshape(9, C, C).astype(bf)
    clsw = jnp.pad(cls_w, ((0, 0), (0, CP - nc))).astype(bf)
    clsb = jnp.pad(cls_b, ((0, 0), (0, CP - nc))).reshape(CP, 1)

    def const(*shape):
        nd = len(shape)
        return pl.BlockSpec(shape, lambda n, _nd=nd: (0,) * _nd)

    out = pl.pallas_call(
        functools.partial(_fused_kernel, H=H, W=W, P=P, dils=_DILATIONS),
        out_shape=jax.ShapeDtypeStruct((N, nc, HW), jnp.float32),
        grid=(N,),
        in_specs=[
            pl.BlockSpec((1, H, W, cin), lambda n: (n, 0, 0, 0)),
            const(cin, C), const(1, C),
            const(9 * len(_DILATIONS), cin, C), const(len(_DILATIONS), 1, C),
            const(cin, C), const(1, C),
            const(5, C, C), const(1, C),
            const(9, C, C), const(1, C),
            const(cin, CP), const(CP, 1),
        ],
        out_specs=pl.BlockSpec((1, nc, HW), lambda n: (n, 0, 0)),
        scratch_shapes=[
            pltpu.VMEM((Hp, 40 + W + P, cin), bf),
            pltpu.VMEM((Hp, 44 + W + P, cin), bf),
            pltpu.VMEM((H + 2, W, C), bf),
            pltpu.VMEM((H + 2, W, C), bf),
            pltpu.VMEM((H + 2, W, C), bf),
        ],
        compiler_params=pltpu.CompilerParams(
            dimension_semantics=("parallel",),
            vmem_limit_bytes=100 * 1024 * 1024),
    )(xh, b0w, b0_offset, dilw, dil_offset, poolw, pool_offset,
      projw, proj_offset, headw, head_offset, clsw, clsb)
    return out.reshape(N, nc, H, W)
